# baseline (speedup 1.0000x reference)
import functools

import numpy as np
import jax
import jax.numpy as jnp
from jax import lax
from jax.experimental import pallas as pl
from jax.experimental.pallas import tpu as pltpu
from jax.experimental.pallas import tpu_sc as plsc

N_SEG = 10000
D = 128
NC = 2
NS = 16
NW = NC * NS
NVW = 2 * NW
CHUNK = 80
NRING = 3
ACC_ROWS = N_SEG + 2 * NS
ZROWS = ACC_ROWS // NS
CPW = 125
A_CHUNKS = 62

_ORDER = []
for _i in range(A_CHUNKS):
    _ORDER.append((_i, 0))
    _ORDER.append((A_CHUNKS + _i, 1))
_ORDER.append((CPW - 1, 1))


def _sc_segment_scatter(atoms, ids2d, zrows, zcnt, ones_hbm, n_chunks):
    mesh = plsc.VectorSubcoreMesh(core_axis_name="c", subcore_axis_name="s")

    @functools.partial(
        pl.kernel,
        out_type=[
            jax.ShapeDtypeStruct((NC, ACC_ROWS, D), jnp.float32),
            jax.ShapeDtypeStruct((NC, ACC_ROWS, 16), jnp.float32),
        ],
        mesh=mesh,
        scratch_types=[
            pltpu.VMEM((NRING, CHUNK, D), jnp.float32),
            pltpu.VMEM((NRING, CHUNK), jnp.int32),
            pltpu.VMEM((CHUNK, 16), jnp.float32),
            pltpu.VMEM_SHARED((ACC_ROWS, D), jnp.float32),
            pltpu.VMEM_SHARED((ACC_ROWS, 16), jnp.float32),
            pltpu.SemaphoreType.DMA((NRING,)),
            pltpu.SemaphoreType.DMA((NRING,)),
        ],
        compiler_params=pltpu.CompilerParams(use_tc_tiling_on_sc=False),
    )
    def body(atoms_hbm, ids_hbm, zrows_hbm, zcnt_hbm, ones_in, sums_out,
             cnts_out, rb, ib, onesbuf, acc, cnt, fsem, ssem):
        cid = lax.axis_index("c")
        sid = lax.axis_index("s")
        wid = cid * NS + sid

        def fetch_copies(p, r):
            c = wid * CPW + _ORDER[p][0]
            return (
                pltpu.make_async_copy(ids_hbm.at[c], ib.at[r], fsem.at[r]),
                pltpu.make_async_copy(
                    atoms_hbm.at[pl.ds(c * CHUNK, CHUNK), :], rb.at[r],
                    fsem.at[r]),
            )

        def start_scatter(p, r):
            q = _ORDER[p][1]
            pltpu.async_copy(rb.at[r], acc.at[ib.at[r]], ssem.at[r],
                             add=True, priority=q)
            pltpu.async_copy(onesbuf, cnt.at[ib.at[r]], ssem.at[r],
                             add=True, priority=q)

        def wait_scatter(r):
            pltpu.make_async_copy(rb.at[r], acc.at[ib.at[r]],
                                  ssem.at[r]).wait()
            pltpu.make_async_copy(onesbuf, cnt.at[ib.at[r]],
                                  ssem.at[r]).wait()

        for p in range(NRING):
            for c in fetch_copies(p, p):
                c.start()

        pltpu.sync_copy(zrows_hbm, acc.at[pl.ds(sid * ZROWS, ZROWS), :])
        pltpu.sync_copy(zcnt_hbm, cnt.at[pl.ds(sid * ZROWS, ZROWS), :])
        pltpu.sync_copy(ones_in, onesbuf)
        plsc.subcore_barrier()

        for p in range(CPW):
            r = p % NRING
            for c in fetch_copies(p, r):
                c.wait()
            start_scatter(p, r)
            if p > 0:
                pr = (p - 1) % NRING
                wait_scatter(pr)
                if p + 2 < CPW:
                    for c in fetch_copies(p + 2, pr):
                        c.start()
        wait_scatter((CPW - 1) % NRING)

        plsc.subcore_barrier()

        out_copies = (
            pltpu.make_async_copy(
                acc.at[pl.ds(sid * ZROWS, ZROWS), :],
                sums_out.at[cid, pl.ds(sid * ZROWS, ZROWS), :], fsem.at[0]),
            pltpu.make_async_copy(
                cnt.at[pl.ds(sid * ZROWS, ZROWS), :],
                cnts_out.at[cid, pl.ds(sid * ZROWS, ZROWS), :], fsem.at[1]),
        )
        for c in out_copies:
            c.start()
        for c in out_copies:
            c.wait()

    return body(atoms, ids2d, zrows, zcnt, ones_hbm)


def _prep(segment_ids, per_w):
    starts = []
    fixvals = []
    counts = []
    for w in range(NW):
        sid = w % NS
        starts += [w * per_w, w * per_w + A_CHUNKS * CHUNK]
        fixvals += [N_SEG + 2 * sid, N_SEG + 2 * sid + 1]
        counts += [A_CHUNKS * CHUNK, per_w - A_CHUNKS * CHUNK]
    starts = np.asarray(starts, np.int32)
    vw_of_atom = np.repeat(np.arange(NVW, dtype=np.int32), counts)
    fix_of_atom = jnp.asarray(np.asarray(fixvals, np.int32)[vw_of_atom])
    leads = segment_ids[jnp.asarray(starts)]
    lead_of_atom = leads[jnp.asarray(vw_of_atom)]
    ids_fix = jnp.where(segment_ids == lead_of_atom, fix_of_atom, segment_ids)
    return ids_fix, leads.reshape(NVW, 1)


def _finalize(sums, cnts, fixs, fixc, leads):
    rows = 2000
    grid = N_SEG // rows

    def fin(s_ref, c_ref, fs_ref, fc_ref, lead_ref, o_ref):
        i = pl.program_id(0)
        base = i * rows
        riota = lax.broadcasted_iota(jnp.int32, (1, rows), 1) + base
        oh = (lead_ref[...] == riota).astype(jnp.float32)
        s = s_ref[0] + s_ref[1]
        s = s + lax.dot_general(oh, fs_ref[...], (((0,), (0,)), ((), ())),
                                preferred_element_type=jnp.float32)
        c = c_ref[0, :, 0:1] + c_ref[1, :, 0:1]
        c = c + lax.dot_general(oh, fc_ref[:, 0:1], (((0,), (0,)), ((), ())),
                                preferred_element_type=jnp.float32)
        o_ref[...] = s / jnp.maximum(c, 1.0)

    return pl.pallas_call(
        fin,
        grid=(grid,),
        in_specs=[
            pl.BlockSpec((NC, rows, D), lambda i: (0, i, 0)),
            pl.BlockSpec((NC, rows, 16), lambda i: (0, i, 0)),
            pl.BlockSpec((NVW, D), lambda i: (0, 0)),
            pl.BlockSpec((NVW, 16), lambda i: (0, 0)),
            pl.BlockSpec((NVW, 1), lambda i: (0, 0)),
        ],
        out_specs=pl.BlockSpec((rows, D), lambda i: (i, 0)),
        out_shape=jax.ShapeDtypeStruct((N_SEG, D), jnp.float32),
    )(sums, cnts, fixs, fixc, leads)


def kernel(atoms, segment_ids, num_segments):
    n_atoms = atoms.shape[0]
    n_chunks = n_atoms // CHUNK
    per_w = n_atoms // NW

    ids_fix, leads = _prep(segment_ids, per_w)
    ids2d = ids_fix.reshape(n_chunks, CHUNK)
    zrows = jnp.zeros((ZROWS, D), jnp.float32)
    zcnt = jnp.zeros((ZROWS, 16), jnp.float32)
    ones_hbm = jnp.ones((CHUNK, 16), jnp.float32)
    sums, cnts = _sc_segment_scatter(atoms, ids2d, zrows, zcnt, ones_hbm,
                                     n_chunks)
    fixs = sums[:, N_SEG:, :].reshape(NVW, D)
    fixc = cnts[:, N_SEG:, :].reshape(NVW, 16)
    return _finalize(sums, cnts, fixs, fixc, leads)


# speedup vs baseline: 1.4199x; 1.4199x over previous
import functools

import jax
import jax.numpy as jnp
from jax import lax
from jax.experimental import pallas as pl
from jax.experimental.pallas import tpu as pltpu
from jax.experimental.pallas import tpu_sc as plsc

N_SEG = 10000
D = 128
NC = 2
NS = 16
NW = NC * NS
CHUNK = 80
NRING = 3
ACC_ROWS = N_SEG + NS
ZROWS = ACC_ROWS // NS


def _sc_segment_scatter(atoms, ids2d, zrows, zcnt, ones_hbm, n_chunks):
    chunks_per_w = n_chunks // NW
    mesh = plsc.VectorSubcoreMesh(core_axis_name="c", subcore_axis_name="s")

    @functools.partial(
        pl.kernel,
        out_type=[
            jax.ShapeDtypeStruct((NC, ACC_ROWS, D), jnp.float32),
            jax.ShapeDtypeStruct((NC, ACC_ROWS, 16), jnp.float32),
        ],
        mesh=mesh,
        scratch_types=[
            pltpu.VMEM((NRING, CHUNK, D), jnp.float32),
            pltpu.VMEM((NRING, CHUNK), jnp.int32),
            pltpu.VMEM((CHUNK, 16), jnp.float32),
            pltpu.VMEM_SHARED((ACC_ROWS, D), jnp.float32),
            pltpu.VMEM_SHARED((ACC_ROWS, 16), jnp.float32),
            pltpu.SemaphoreType.DMA((NRING,)),
            pltpu.SemaphoreType.DMA((NRING,)),
        ],
        compiler_params=pltpu.CompilerParams(use_tc_tiling_on_sc=False),
    )
    def body(atoms_hbm, ids_hbm, zrows_hbm, zcnt_hbm, ones_in, sums_out,
             cnts_out, rb, ib, onesbuf, acc, cnt, fsem, ssem):
        cid = lax.axis_index("c")
        sid = lax.axis_index("s")
        wid = cid * NS + sid

        def fetch_copies(b, r):
            c = wid * chunks_per_w + b
            return (
                pltpu.make_async_copy(ids_hbm.at[c], ib.at[r], fsem.at[r]),
                pltpu.make_async_copy(
                    atoms_hbm.at[pl.ds(c * CHUNK, CHUNK), :], rb.at[r],
                    fsem.at[r]),
            )

        def start_scatter(r):
            pltpu.async_copy(rb.at[r], acc.at[ib.at[r]], ssem.at[r], add=True)
            pltpu.async_copy(onesbuf, cnt.at[ib.at[r]], ssem.at[r], add=True,
                             priority=1)

        def wait_scatter(r):
            pltpu.make_async_copy(rb.at[r], acc.at[ib.at[r]],
                                  ssem.at[r]).wait()
            pltpu.make_async_copy(onesbuf, cnt.at[ib.at[r]],
                                  ssem.at[r]).wait()

        for b in range(NRING):
            for c in fetch_copies(b, b):
                c.start()

        pltpu.sync_copy(zrows_hbm, acc.at[pl.ds(sid * ZROWS, ZROWS), :])
        pltpu.sync_copy(zcnt_hbm, cnt.at[pl.ds(sid * ZROWS, ZROWS), :])
        pltpu.sync_copy(ones_in, onesbuf)
        plsc.subcore_barrier()

        for b in range(chunks_per_w):
            r = b % NRING
            for c in fetch_copies(b, r):
                c.wait()
            start_scatter(r)
            wait_scatter(r)
            nxt = b + NRING
            if nxt < chunks_per_w:
                for c in fetch_copies(nxt, r):
                    c.start()

        plsc.subcore_barrier()

        out_copies = (
            pltpu.make_async_copy(
                acc.at[pl.ds(sid * ZROWS, ZROWS), :],
                sums_out.at[cid, pl.ds(sid * ZROWS, ZROWS), :], fsem.at[0]),
            pltpu.make_async_copy(
                cnt.at[pl.ds(sid * ZROWS, ZROWS), :],
                cnts_out.at[cid, pl.ds(sid * ZROWS, ZROWS), :], fsem.at[1]),
        )
        for c in out_copies:
            c.start()
        for c in out_copies:
            c.wait()

    return body(atoms, ids2d, zrows, zcnt, ones_hbm)


def _prep(segment_ids, per_w):
    n_atoms = segment_ids.shape[0]
    leads = segment_ids[::per_w]
    sid_of_atom = (jnp.arange(n_atoms, dtype=jnp.int32) // per_w) % NS
    lead_of_atom = jnp.repeat(leads, per_w)
    ids_fix = jnp.where(segment_ids == lead_of_atom,
                        N_SEG + sid_of_atom, segment_ids)
    return ids_fix, leads.reshape(NW, 1)


def _finalize(sums, cnts, fixs, fixc, leads):
    rows = 2000
    grid = N_SEG // rows

    def fin(s_ref, c_ref, fs_ref, fc_ref, lead_ref, o_ref):
        i = pl.program_id(0)
        base = i * rows
        riota = lax.broadcasted_iota(jnp.int32, (1, rows), 1) + base
        oh = (lead_ref[...] == riota).astype(jnp.float32)
        s = s_ref[0] + s_ref[1]
        s = s + lax.dot_general(oh, fs_ref[...], (((0,), (0,)), ((), ())),
                                preferred_element_type=jnp.float32)
        c = c_ref[0, :, 0:1] + c_ref[1, :, 0:1]
        c = c + lax.dot_general(oh, fc_ref[:, 0:1], (((0,), (0,)), ((), ())),
                                preferred_element_type=jnp.float32)
        o_ref[...] = s / jnp.maximum(c, 1.0)

    return pl.pallas_call(
        fin,
        grid=(grid,),
        in_specs=[
            pl.BlockSpec((NC, rows, D), lambda i: (0, i, 0)),
            pl.BlockSpec((NC, rows, 16), lambda i: (0, i, 0)),
            pl.BlockSpec((NW, D), lambda i: (0, 0)),
            pl.BlockSpec((NW, 16), lambda i: (0, 0)),
            pl.BlockSpec((NW, 1), lambda i: (0, 0)),
        ],
        out_specs=pl.BlockSpec((rows, D), lambda i: (i, 0)),
        out_shape=jax.ShapeDtypeStruct((N_SEG, D), jnp.float32),
    )(sums, cnts, fixs, fixc, leads)


def kernel(atoms, segment_ids, num_segments):
    n_atoms = atoms.shape[0]
    n_chunks = n_atoms // CHUNK
    per_w = n_atoms // NW

    ids_fix, leads = _prep(segment_ids, per_w)
    ids2d = ids_fix.reshape(n_chunks, CHUNK)
    zrows = jnp.zeros((ZROWS, D), jnp.float32)
    zcnt = jnp.zeros((ZROWS, 16), jnp.float32)
    ones_hbm = jnp.ones((CHUNK, 16), jnp.float32)
    sums, cnts = _sc_segment_scatter(atoms, ids2d, zrows, zcnt, ones_hbm,
                                     n_chunks)
    fixs = sums[:, N_SEG:, :].reshape(NW, D)
    fixc = cnts[:, N_SEG:, :].reshape(NW, 16)
    return _finalize(sums, cnts, fixs, fixc, leads)


# speedup vs baseline: 1.4274x; 1.0053x over previous
import functools

import jax
import jax.numpy as jnp
from jax import lax
from jax.experimental import pallas as pl
from jax.experimental.pallas import tpu as pltpu
from jax.experimental.pallas import tpu_sc as plsc

N_SEG = 10000
D = 128
NC = 2
NS = 16
NW = NC * NS
CHUNK = 80
NRING = 3
ACC_ROWS = N_SEG + NS
ZROWS = ACC_ROWS // NS


def _sc_segment_scatter(atoms, ids2d, zrows, zcnt, ones_hbm, n_chunks):
    chunks_per_w = n_chunks // NW
    mesh = plsc.VectorSubcoreMesh(core_axis_name="c", subcore_axis_name="s")

    @functools.partial(
        pl.kernel,
        out_type=[
            jax.ShapeDtypeStruct((NC, ACC_ROWS, D), jnp.float32),
            jax.ShapeDtypeStruct((NC, ACC_ROWS, 16), jnp.float32),
        ],
        mesh=mesh,
        scratch_types=[
            pltpu.VMEM((NRING, CHUNK, D), jnp.float32),
            pltpu.VMEM((NRING, CHUNK), jnp.int32),
            pltpu.VMEM((CHUNK, 16), jnp.float32),
            pltpu.VMEM_SHARED((ACC_ROWS, D), jnp.float32),
            pltpu.VMEM_SHARED((ACC_ROWS, 16), jnp.float32),
            pltpu.SemaphoreType.DMA((NRING,)),
            pltpu.SemaphoreType.DMA((NRING,)),
        ],
        compiler_params=pltpu.CompilerParams(use_tc_tiling_on_sc=False),
    )
    def body(atoms_hbm, ids_hbm, zrows_hbm, zcnt_hbm, ones_in, sums_out,
             cnts_out, rb, ib, onesbuf, acc, cnt, fsem, ssem):
        cid = lax.axis_index("c")
        sid = lax.axis_index("s")
        wid = cid * NS + sid

        def fetch_copies(b, r):
            c = wid * chunks_per_w + b
            return (
                pltpu.make_async_copy(ids_hbm.at[c], ib.at[r], fsem.at[r]),
                pltpu.make_async_copy(
                    atoms_hbm.at[pl.ds(c * CHUNK, CHUNK), :], rb.at[r],
                    fsem.at[r]),
            )

        def start_scatter(r):
            pltpu.async_copy(rb.at[r], acc.at[ib.at[r]], ssem.at[r], add=True)
            pltpu.async_copy(onesbuf, cnt.at[ib.at[r]], ssem.at[r], add=True,
                             priority=1)

        def wait_scatter(r):
            pltpu.make_async_copy(rb.at[r], acc.at[ib.at[r]],
                                  ssem.at[r]).wait()
            pltpu.make_async_copy(onesbuf, cnt.at[ib.at[r]],
                                  ssem.at[r]).wait()

        for b in range(NRING):
            for c in fetch_copies(b, b):
                c.start()

        pltpu.sync_copy(zrows_hbm, acc.at[pl.ds(sid * ZROWS, ZROWS), :])
        pltpu.sync_copy(zcnt_hbm, cnt.at[pl.ds(sid * ZROWS, ZROWS), :])
        pltpu.sync_copy(ones_in, onesbuf)
        plsc.subcore_barrier()

        for b in range(chunks_per_w):
            r = b % NRING
            for c in fetch_copies(b, r):
                c.wait()
            start_scatter(r)
            wait_scatter(r)
            nxt = b + NRING
            if nxt < chunks_per_w:
                for c in fetch_copies(nxt, r):
                    c.start()

        plsc.subcore_barrier()

        out_copies = (
            pltpu.make_async_copy(
                acc.at[pl.ds(sid * ZROWS, ZROWS), :],
                sums_out.at[cid, pl.ds(sid * ZROWS, ZROWS), :], fsem.at[0]),
            pltpu.make_async_copy(
                cnt.at[pl.ds(sid * ZROWS, ZROWS), :],
                cnts_out.at[cid, pl.ds(sid * ZROWS, ZROWS), :], fsem.at[1]),
        )
        for c in out_copies:
            c.start()
        for c in out_copies:
            c.wait()

    return body(atoms, ids2d, zrows, zcnt, ones_hbm)


def _prep(segment_ids, per_w):
    n_atoms = segment_ids.shape[0]
    leads = segment_ids[::per_w]
    sid_of_atom = (jnp.arange(n_atoms, dtype=jnp.int32) // per_w) % NS
    lead_of_atom = jnp.repeat(leads, per_w)
    ids_fix = jnp.where(segment_ids == lead_of_atom,
                        N_SEG + sid_of_atom, segment_ids)
    return ids_fix, leads.reshape(NW, 1)


def _finalize(sums, cnts, fixs, fixc, leads):
    rows = 5000
    grid = N_SEG // rows

    def fin(s_ref, c_ref, fs_ref, fc_ref, lead_ref, o_ref):
        i = pl.program_id(0)
        base = i * rows
        riota = lax.broadcasted_iota(jnp.int32, (1, rows), 1) + base
        oh = (lead_ref[...] == riota).astype(jnp.float32)
        s = s_ref[0] + s_ref[1]
        s = s + lax.dot_general(oh, fs_ref[...], (((0,), (0,)), ((), ())),
                                preferred_element_type=jnp.float32)
        c = c_ref[0, :, 0:1] + c_ref[1, :, 0:1]
        c = c + lax.dot_general(oh, fc_ref[:, 0:1], (((0,), (0,)), ((), ())),
                                preferred_element_type=jnp.float32)
        o_ref[...] = s / jnp.maximum(c, 1.0)

    return pl.pallas_call(
        fin,
        grid=(grid,),
        in_specs=[
            pl.BlockSpec((NC, rows, D), lambda i: (0, i, 0)),
            pl.BlockSpec((NC, rows, 16), lambda i: (0, i, 0)),
            pl.BlockSpec((NW, D), lambda i: (0, 0)),
            pl.BlockSpec((NW, 16), lambda i: (0, 0)),
            pl.BlockSpec((NW, 1), lambda i: (0, 0)),
        ],
        out_specs=pl.BlockSpec((rows, D), lambda i: (i, 0)),
        out_shape=jax.ShapeDtypeStruct((N_SEG, D), jnp.float32),
    )(sums, cnts, fixs, fixc, leads)


def kernel(atoms, segment_ids, num_segments):
    n_atoms = atoms.shape[0]
    n_chunks = n_atoms // CHUNK
    per_w = n_atoms // NW

    ids_fix, leads = _prep(segment_ids, per_w)
    ids2d = ids_fix.reshape(n_chunks, CHUNK)
    zrows = jnp.zeros((ZROWS, D), jnp.float32)
    zcnt = jnp.zeros((ZROWS, 16), jnp.float32)
    ones_hbm = jnp.ones((CHUNK, 16), jnp.float32)
    sums, cnts = _sc_segment_scatter(atoms, ids2d, zrows, zcnt, ones_hbm,
                                     n_chunks)
    fixs = sums[:, N_SEG:, :].reshape(NW, D)
    fixc = cnts[:, N_SEG:, :].reshape(NW, 16)
    return _finalize(sums, cnts, fixs, fixc, leads)
